# baseline (speedup 1.0000x reference)
import jax
import jax.numpy as jnp
from jax import lax
from jax.experimental import pallas as pl
from jax.experimental.pallas import tpu as pltpu
from jax.experimental.pallas import tpu_sc as plsc

_N = 1000000
_BATCH = 4096
_EMB = 64
_DECAY = 1e-4
_NW = 32
_BPW = _BATCH // _NW
_R = 256
_T = _BATCH // _R


def _gather_body(ut, it, users, pos, out_u, out_p,
                 uidx, umat, ubuf, pidx, pmat, pbuf, sem_u, sem_p):
    wid = lax.axis_index("s") * 2 + lax.axis_index("c")
    base = wid * _BPW
    pltpu.sync_copy(users.at[pl.ds(base, _BPW)], uidx)
    pltpu.sync_copy(pos.at[pl.ds(base, _BPW)], pidx)

    def build(r, _):
        for i in range(_BPW // 16):
            s = pl.ds(i * 16, 16)
            off = jnp.broadcast_to(r * _N, (16,))
            umat[r, s] = uidx[s] + off
            pmat[r, s] = pidx[s] + off
        return 0
    lax.fori_loop(0, _EMB, build, 0, unroll=False)

    def fire(r, _):
        pltpu.async_copy(ut.at[umat.at[r]], ubuf.at[r], sem_u)
        pltpu.async_copy(it.at[pmat.at[r]], pbuf.at[r], sem_p)
        return 0
    lax.fori_loop(0, _EMB, fire, 0, unroll=False)

    def drain(r, _):
        pltpu.make_async_copy(ut.at[umat.at[r]], ubuf.at[r], sem_u).wait()
        pltpu.make_async_copy(it.at[pmat.at[r]], pbuf.at[r], sem_p).wait()
        return 0
    lax.fori_loop(0, _EMB, drain, 0, unroll=False)

    pltpu.sync_copy(ubuf, out_u.at[:, pl.ds(base, _BPW)])
    pltpu.sync_copy(pbuf, out_p.at[:, pl.ds(base, _BPW)])


def _loss_body(tu_ref, tp_ref, acc_ref, un_ref, pn_ref):
    t = pl.program_id(0)

    @pl.when(t == 0)
    def _init():
        ut = tu_ref[...]
        pt = tp_ref[...]
        usq = jnp.sum(ut * ut, axis=0, keepdims=True)
        psq = jnp.sum(pt * pt, axis=0, keepdims=True)
        unt = ut / jnp.sqrt(usq)
        pnt = pt / jnp.sqrt(psq)
        un_ref[...] = unt.astype(jnp.bfloat16)
        pn_ref[...] = pnt.astype(jnp.bfloat16)
        acc_ref[0] = jnp.sum((unt - pnt) ** 2)
        acc_ref[1] = jnp.sum(usq) + jnp.sum(psq)
        acc_ref[2] = jnp.float32(-_BATCH)
        acc_ref[3] = jnp.float32(-_BATCH)

    dn = (((0,), (0,)), ((), ()))
    gu = lax.dot_general(un_ref[:, pl.ds(t * _R, _R)], un_ref[...], dn,
                         preferred_element_type=jnp.float32)
    gp = lax.dot_general(pn_ref[:, pl.ds(t * _R, _R)], pn_ref[...], dn,
                         preferred_element_type=jnp.float32)
    acc_ref[2] += jnp.sum(jnp.exp(jnp.minimum(4.0 * gu - 4.0, 0.0)))
    acc_ref[3] += jnp.sum(jnp.exp(jnp.minimum(4.0 * gp - 4.0, 0.0)))


def kernel(user_embed, item_embed, users, pos_items):
    t1u = user_embed.T.reshape(-1)
    t1p = item_embed.T.reshape(-1)

    gather = pl.kernel(
        _gather_body,
        mesh=plsc.VectorSubcoreMesh(core_axis_name="c", subcore_axis_name="s"),
        out_type=[jax.ShapeDtypeStruct((_EMB, _BATCH), jnp.float32),
                  jax.ShapeDtypeStruct((_EMB, _BATCH), jnp.float32)],
        scratch_types=[
            pltpu.VMEM((_BPW,), jnp.int32),
            pltpu.VMEM((_EMB, _BPW), jnp.int32),
            pltpu.VMEM((_EMB, _BPW), jnp.float32),
            pltpu.VMEM((_BPW,), jnp.int32),
            pltpu.VMEM((_EMB, _BPW), jnp.int32),
            pltpu.VMEM((_EMB, _BPW), jnp.float32),
            pltpu.SemaphoreType.DMA,
            pltpu.SemaphoreType.DMA,
        ],
        compiler_params=pltpu.CompilerParams(use_tc_tiling_on_sc=False),
    )
    tu, tp = gather(t1u, t1p, users, pos_items)

    acc = pl.pallas_call(
        _loss_body,
        grid=(_T,),
        in_specs=[pl.BlockSpec((_EMB, _BATCH), lambda t: (0, 0)),
                  pl.BlockSpec((_EMB, _BATCH), lambda t: (0, 0))],
        out_specs=pl.BlockSpec((4,), lambda t: (0,), memory_space=pltpu.SMEM),
        out_shape=jax.ShapeDtypeStruct((4,), jnp.float32),
        scratch_shapes=[
            pltpu.VMEM((_EMB, _BATCH), jnp.bfloat16),
            pltpu.VMEM((_EMB, _BATCH), jnp.bfloat16),
        ],
    )(tu, tp)

    n_pairs = _BATCH * (_BATCH - 1) / 2.0
    align = acc[0] / _BATCH
    uniformity = 0.5 * (jnp.log(acc[2] * (0.5 / n_pairs))
                        + jnp.log(acc[3] * (0.5 / n_pairs)))
    emb_loss = (_DECAY * 0.5 / _BATCH) * acc[1]
    return align + uniformity + emb_loss, emb_loss


# speedup vs baseline: 8.6550x; 8.6550x over previous
import jax
import jax.numpy as jnp
from jax import lax
from jax.experimental import pallas as pl
from jax.experimental.pallas import tpu as pltpu
from jax.experimental.pallas import tpu_sc as plsc

_BATCH = 4096
_EMB = 64
_WIDE = 2 * _EMB
_DECAY = 1e-4
_NW = 32
_BPW = _BATCH // _NW
_R = 256
_T = _BATCH // _R


def _gather_body(ut, it, users, pos, out_u, out_p,
                 uidx, uhalf, urows, pidx, phalf, prows, sem_u, sem_p):
    wid = lax.axis_index("s") * 2 + lax.axis_index("c")
    base = wid * _BPW
    pltpu.sync_copy(users.at[pl.ds(base, _BPW)], uidx)
    pltpu.sync_copy(pos.at[pl.ds(base, _BPW)], pidx)
    for i in range(_BPW // 16):
        s = pl.ds(i * 16, 16)
        uhalf[s] = lax.shift_right_logical(uidx[s], 1)
        phalf[s] = lax.shift_right_logical(pidx[s], 1)
    cu = pltpu.async_copy(ut.at[uhalf], urows, sem_u)
    cp = pltpu.async_copy(it.at[phalf], prows, sem_p)
    cu.wait()
    cp.wait()
    pltpu.sync_copy(urows, out_u.at[pl.ds(base, _BPW)])
    pltpu.sync_copy(prows, out_p.at[pl.ds(base, _BPW)])


def _loss_body(wu_ref, wp_ref, users_ref, pos_ref, acc_ref, un_ref, pn_ref):
    t = pl.program_id(0)

    @pl.when(t == 0)
    def _init():
        u_odd = lax.rem(users_ref[...], 2) == 1
        p_odd = lax.rem(pos_ref[...], 2) == 1
        ug = jnp.where(u_odd, wu_ref[:, _EMB:], wu_ref[:, :_EMB])
        pg = jnp.where(p_odd, wp_ref[:, _EMB:], wp_ref[:, :_EMB])
        usq = jnp.sum(ug * ug, axis=1, keepdims=True)
        psq = jnp.sum(pg * pg, axis=1, keepdims=True)
        un = ug / jnp.sqrt(usq)
        pn = pg / jnp.sqrt(psq)
        un_ref[...] = un.astype(jnp.bfloat16)
        pn_ref[...] = pn.astype(jnp.bfloat16)
        acc_ref[0] = jnp.sum((un - pn) ** 2)
        acc_ref[1] = jnp.sum(usq) + jnp.sum(psq)
        acc_ref[2] = jnp.float32(-_BATCH)
        acc_ref[3] = jnp.float32(-_BATCH)

    dn = (((1,), (1,)), ((), ()))
    gu = lax.dot_general(un_ref[pl.ds(t * _R, _R), :], un_ref[...], dn,
                         preferred_element_type=jnp.float32)
    gp = lax.dot_general(pn_ref[pl.ds(t * _R, _R), :], pn_ref[...], dn,
                         preferred_element_type=jnp.float32)
    acc_ref[2] += jnp.sum(jnp.exp(jnp.minimum(4.0 * gu - 4.0, 0.0)))
    acc_ref[3] += jnp.sum(jnp.exp(jnp.minimum(4.0 * gp - 4.0, 0.0)))


def kernel(user_embed, item_embed, users, pos_items):
    ut2 = user_embed.reshape(-1, _WIDE)
    it2 = item_embed.reshape(-1, _WIDE)

    gather = pl.kernel(
        _gather_body,
        mesh=plsc.VectorSubcoreMesh(core_axis_name="c", subcore_axis_name="s"),
        out_type=[jax.ShapeDtypeStruct((_BATCH, _WIDE), jnp.float32),
                  jax.ShapeDtypeStruct((_BATCH, _WIDE), jnp.float32)],
        scratch_types=[
            pltpu.VMEM((_BPW,), jnp.int32),
            pltpu.VMEM((_BPW,), jnp.int32),
            pltpu.VMEM((_BPW, _WIDE), jnp.float32),
            pltpu.VMEM((_BPW,), jnp.int32),
            pltpu.VMEM((_BPW,), jnp.int32),
            pltpu.VMEM((_BPW, _WIDE), jnp.float32),
            pltpu.SemaphoreType.DMA,
            pltpu.SemaphoreType.DMA,
        ],
    )
    wu, wp = gather(ut2, it2, users, pos_items)

    acc = pl.pallas_call(
        _loss_body,
        grid=(_T,),
        in_specs=[pl.BlockSpec((_BATCH, _WIDE), lambda t: (0, 0)),
                  pl.BlockSpec((_BATCH, _WIDE), lambda t: (0, 0)),
                  pl.BlockSpec((_BATCH, 1), lambda t: (0, 0)),
                  pl.BlockSpec((_BATCH, 1), lambda t: (0, 0))],
        out_specs=pl.BlockSpec((4,), lambda t: (0,), memory_space=pltpu.SMEM),
        out_shape=jax.ShapeDtypeStruct((4,), jnp.float32),
        scratch_shapes=[
            pltpu.VMEM((_BATCH, _EMB), jnp.bfloat16),
            pltpu.VMEM((_BATCH, _EMB), jnp.bfloat16),
        ],
    )(wu, wp, users.reshape(_BATCH, 1), pos_items.reshape(_BATCH, 1))

    n_pairs = _BATCH * (_BATCH - 1) / 2.0
    align = acc[0] / _BATCH
    uniformity = 0.5 * (jnp.log(acc[2] * (0.5 / n_pairs))
                        + jnp.log(acc[3] * (0.5 / n_pairs)))
    emb_loss = (_DECAY * 0.5 / _BATCH) * acc[1]
    return align + uniformity + emb_loss, emb_loss
